# Initial kernel scaffold; baseline (speedup 1.0000x reference)
import jax
import jax.numpy as jnp
from jax.experimental import pallas as pl


def kernel(x, emb_table, fc_w, fc_b, fc2_w, fc2_b, conv1_wl, conv1_bl, conv1_wr, conv2_wl, conv2_bl, conv2_wr, lin_w, lin_b, x0, edge_0):
    raise NotImplementedError("write your pallas kernel here")


# speedup vs baseline: 7.2897x; 7.2897x over previous
import functools

import jax
import jax.numpy as jnp
from jax import lax
from jax.experimental import pallas as pl

B = 4096
N0 = 3190
E0 = 51040

NP = 3328
CH = 256
EP = 51200
NCH = EP // CH
R = 512


def _seg_kernel(x0p_ref, src_ref, dst_ref, w1l_ref, b1l_ref, w1r_ref, out_ref):
    x0p = x0p_ref[...]
    row = lax.broadcasted_iota(jnp.int32, (NP, CH), 0)

    def body(i, acc):
        s = src_ref[i]
        d = dst_ref[i]
        ost = (s == row).astype(jnp.float32)
        g = lax.dot_general(
            ost, x0p, (((0,), (0,)), ((), ())),
            preferred_element_type=jnp.float32)
        odt = (d == row).astype(jnp.float32)
        acc = acc + jnp.dot(odt, g, preferred_element_type=jnp.float32)
        return acc

    acc = lax.fori_loop(0, NCH, body, jnp.zeros((NP, 16), jnp.float32))
    cnt = acc[:, 10:11]
    mean = acc / jnp.maximum(cnt, 1.0)
    out_ref[...] = jnp.maximum(
        jnp.dot(mean, w1l_ref[...], preferred_element_type=jnp.float32)
        + b1l_ref[...]
        + jnp.dot(x0p, w1r_ref[...], preferred_element_type=jnp.float32),
        0.0)


def _main_kernel(x2_ref, w1_ref, w2_ref, b2_ref, w3_ref, b3_ref, x0t_ref,
                 out1_ref, w1r_ref, b1l_ref, w2l_ref, b2l_ref, w2r_ref,
                 lin_ref, linb_ref, out_ref):
    x2 = x2_ref[...]
    h1 = jnp.maximum(
        jnp.dot(x2, w1_ref[...], preferred_element_type=jnp.float32), 0.0)
    h2 = jnp.maximum(
        jnp.dot(h1, w2_ref[...], preferred_element_type=jnp.float32)
        + b2_ref[...], 0.0)
    h = jnp.maximum(
        jnp.dot(h2, w3_ref[...], preferred_element_type=jnp.float32)
        + b3_ref[...], 0.0)

    x0t = x0t_ref[...]
    a = jnp.dot(h, x0t, preferred_element_type=jnp.float32)
    norm = jnp.sqrt(jnp.sum(h * h, axis=1, keepdims=True))
    norm0 = jnp.sqrt(jnp.sum(x0t * x0t, axis=0, keepdims=True))
    a = a / norm / norm0
    col = lax.broadcasted_iota(jnp.int32, (R, NP), 1)
    neg = jnp.float32(-jnp.inf)
    a = jnp.where(col < N0, a, neg)
    a = jnp.where(jnp.isnan(a), jnp.float32(jnp.inf), a)

    idxs = []
    v0 = None
    for k in range(6):
        v = jnp.max(a, axis=1, keepdims=True)
        if k == 0:
            v0 = v
        imax = jnp.min(jnp.where(a == v, col, NP), axis=1, keepdims=True)
        idxs.append(imax)
        a = jnp.where(col == imax, neg, a)

    sel = v0 == 1.0
    s_mat = jnp.zeros((R, NP), jnp.float32)
    for k in range(5):
        pick = jnp.where(sel, idxs[k + 1], idxs[k])
        s_mat = s_mat + (col == pick).astype(jnp.float32)

    msum = jnp.dot(s_mat, out1_ref[...], preferred_element_type=jnp.float32)
    mean5 = msum / 5.0

    h1h = jnp.maximum(
        b1l_ref[...]
        + jnp.dot(h, w1r_ref[...], preferred_element_type=jnp.float32), 0.0)

    o2 = (jnp.dot(mean5, w2l_ref[...], preferred_element_type=jnp.float32)
          + b2l_ref[...]
          + jnp.dot(h1h, w2r_ref[...], preferred_element_type=jnp.float32))
    z = (jnp.dot(o2, lin_ref[...], preferred_element_type=jnp.float32)
         + linb_ref[...])
    zm = jnp.max(z, axis=1, keepdims=True)
    e = jnp.exp(z - zm)
    out_ref[...] = e / jnp.sum(e, axis=1, keepdims=True)


def _pad2(m, rows, cols):
    return jnp.zeros((rows, cols), jnp.float32).at[:m.shape[0], :m.shape[1]].set(m)


@jax.jit
def kernel(x, emb_table, fc_w, fc_b, fc2_w, fc2_b, conv1_wl, conv1_bl,
           conv1_wr, conv2_wl, conv2_bl, conv2_wr, lin_w, lin_b, x0, edge_0):
    x2 = jnp.transpose(x, (1, 0, 2)).reshape(B, 300)
    w1 = jnp.kron(jnp.eye(60, dtype=jnp.float32), emb_table)
    w2 = (fc_w.T[:, None, :, None]
          * jnp.eye(5, dtype=jnp.float32)[None, :, None, :]).reshape(300, 100)
    b2 = jnp.repeat(fc_b, 5)[None, :]
    w3 = _pad2(fc2_w.T, 100, 16)
    b3 = _pad2(fc2_b[None, :], 1, 16)

    x0p = _pad2(x0, NP, 16).at[:, 10].set(1.0)
    x0t = _pad2(x0.T, 16, NP)

    pad_e = jnp.full((2, EP - E0), NP - 1, dtype=edge_0.dtype)
    ep = jnp.concatenate([edge_0, pad_e], axis=1)
    srcp = ep[0].reshape(NCH, 1, CH)
    dstp = ep[1].reshape(NCH, 1, CH)

    w1l = _pad2(conv1_wl.T, 16, 16)
    b1l = _pad2(conv1_bl[None, :], 1, 16)
    w1r = _pad2(conv1_wr.T, 16, 16)
    w2l = _pad2(conv2_wl.T, 16, 16)
    b2l = _pad2(conv2_bl[None, :], 1, 16)
    w2r = _pad2(conv2_wr.T, 16, 16)
    lin = _pad2(lin_w.T, 16, 8)
    linb = jnp.full((1, 8), -1e30, jnp.float32).at[0, :3].set(lin_b)

    out1 = pl.pallas_call(
        _seg_kernel,
        out_shape=jax.ShapeDtypeStruct((NP, 16), jnp.float32),
    )(x0p, srcp, dstp, w1l, b1l, w1r)

    full = lambda shape: pl.BlockSpec(shape, lambda i: (0,) * len(shape))
    out = pl.pallas_call(
        _main_kernel,
        grid=(B // R,),
        in_specs=[
            pl.BlockSpec((R, 300), lambda i: (i, 0)),
            full((300, 300)), full((300, 100)), full((1, 100)),
            full((100, 16)), full((1, 16)), full((16, NP)), full((NP, 16)),
            full((16, 16)), full((1, 16)), full((16, 16)), full((1, 16)),
            full((16, 16)), full((16, 8)), full((1, 8)),
        ],
        out_specs=pl.BlockSpec((R, 8), lambda i: (i, 0)),
        out_shape=jax.ShapeDtypeStruct((B, 8), jnp.float32),
    )(x2, w1, w2, b2, w3, b3, x0t, out1,
      w1r, b1l, w2l, b2l, w2r, lin, linb)

    return out[:, :3]


# speedup vs baseline: 8.3302x; 1.1427x over previous
import functools

import jax
import jax.numpy as jnp
from jax import lax
from jax.experimental import pallas as pl

B = 4096
N0 = 3190
E0 = 51040

NP = 3328
CH = 1024
EP = 51200
NCH = EP // CH
R = 512


def _seg_kernel(x0p_ref, src_ref, dst_ref, w1l_ref, b1l_ref, w1r_ref, out_ref):
    x0p = x0p_ref[...]
    row = lax.broadcasted_iota(jnp.int32, (NP, CH), 0)

    def body(i, acc):
        s = src_ref[i]
        d = dst_ref[i]
        ost = (s == row).astype(jnp.float32)
        g = lax.dot_general(
            ost, x0p, (((0,), (0,)), ((), ())),
            preferred_element_type=jnp.float32)
        odt = (d == row).astype(jnp.float32)
        acc = acc + jnp.dot(odt, g, preferred_element_type=jnp.float32)
        return acc

    acc = lax.fori_loop(0, NCH, body, jnp.zeros((NP, 16), jnp.float32))
    cnt = acc[:, 10:11]
    mean = acc / jnp.maximum(cnt, 1.0)
    out_ref[...] = jnp.maximum(
        jnp.dot(mean, w1l_ref[...], preferred_element_type=jnp.float32)
        + b1l_ref[...]
        + jnp.dot(x0p, w1r_ref[...], preferred_element_type=jnp.float32),
        0.0)


def _main_kernel(x2_ref, w1_ref, w2_ref, b2_ref, w3_ref, b3_ref, x0t_ref,
                 out1_ref, w1r_ref, b1l_ref, w2l_ref, b2l_ref, w2r_ref,
                 lin_ref, linb_ref, out_ref):
    x2 = x2_ref[...]
    h1 = jnp.maximum(
        jnp.dot(x2, w1_ref[...], preferred_element_type=jnp.float32), 0.0)
    h2 = jnp.maximum(
        jnp.dot(h1, w2_ref[...], preferred_element_type=jnp.float32)
        + b2_ref[...], 0.0)
    h = jnp.maximum(
        jnp.dot(h2, w3_ref[...], preferred_element_type=jnp.float32)
        + b3_ref[...], 0.0)

    x0t = x0t_ref[...]
    a = jnp.dot(h, x0t, preferred_element_type=jnp.float32)
    norm = jnp.sqrt(jnp.sum(h * h, axis=1, keepdims=True))
    norm0 = jnp.sqrt(jnp.sum(x0t * x0t, axis=0, keepdims=True))
    a = a / norm / norm0
    col = lax.broadcasted_iota(jnp.int32, (R, NP), 1)
    neg = jnp.float32(-jnp.inf)
    a = jnp.where(col < N0, a, neg)
    a = jnp.where(jnp.isnan(a), jnp.float32(jnp.inf), a)

    idxs = []
    v0 = None
    for k in range(6):
        v = jnp.max(a, axis=1, keepdims=True)
        if k == 0:
            v0 = v
        imax = jnp.min(jnp.where(a == v, col, NP), axis=1, keepdims=True)
        idxs.append(imax)
        a = jnp.where(col == imax, neg, a)

    sel = v0 == 1.0
    s_mat = jnp.zeros((R, NP), jnp.float32)
    for k in range(5):
        pick = jnp.where(sel, idxs[k + 1], idxs[k])
        s_mat = s_mat + (col == pick).astype(jnp.float32)

    msum = jnp.dot(s_mat, out1_ref[...], preferred_element_type=jnp.float32)
    mean5 = msum / 5.0

    h1h = jnp.maximum(
        b1l_ref[...]
        + jnp.dot(h, w1r_ref[...], preferred_element_type=jnp.float32), 0.0)

    o2 = (jnp.dot(mean5, w2l_ref[...], preferred_element_type=jnp.float32)
          + b2l_ref[...]
          + jnp.dot(h1h, w2r_ref[...], preferred_element_type=jnp.float32))
    z = (jnp.dot(o2, lin_ref[...], preferred_element_type=jnp.float32)
         + linb_ref[...])
    zm = jnp.max(z, axis=1, keepdims=True)
    e = jnp.exp(z - zm)
    out_ref[...] = e / jnp.sum(e, axis=1, keepdims=True)


def _pad2(m, rows, cols):
    return jnp.zeros((rows, cols), jnp.float32).at[:m.shape[0], :m.shape[1]].set(m)


@jax.jit
def kernel(x, emb_table, fc_w, fc_b, fc2_w, fc2_b, conv1_wl, conv1_bl,
           conv1_wr, conv2_wl, conv2_bl, conv2_wr, lin_w, lin_b, x0, edge_0):
    x2 = jnp.transpose(x, (1, 0, 2)).reshape(B, 300)
    w1 = jnp.kron(jnp.eye(60, dtype=jnp.float32), emb_table)
    w2 = (fc_w.T[:, None, :, None]
          * jnp.eye(5, dtype=jnp.float32)[None, :, None, :]).reshape(300, 100)
    b2 = jnp.repeat(fc_b, 5)[None, :]
    w3 = _pad2(fc2_w.T, 100, 16)
    b3 = _pad2(fc2_b[None, :], 1, 16)

    x0p = _pad2(x0, NP, 16).at[:, 10].set(1.0)
    x0t = _pad2(x0.T, 16, NP)

    pad_e = jnp.full((2, EP - E0), NP - 1, dtype=edge_0.dtype)
    ep = jnp.concatenate([edge_0, pad_e], axis=1)
    srcp = ep[0].reshape(NCH, 1, CH)
    dstp = ep[1].reshape(NCH, 1, CH)

    w1l = _pad2(conv1_wl.T, 16, 16)
    b1l = _pad2(conv1_bl[None, :], 1, 16)
    w1r = _pad2(conv1_wr.T, 16, 16)
    w2l = _pad2(conv2_wl.T, 16, 16)
    b2l = _pad2(conv2_bl[None, :], 1, 16)
    w2r = _pad2(conv2_wr.T, 16, 16)
    lin = _pad2(lin_w.T, 16, 8)
    linb = jnp.full((1, 8), -1e30, jnp.float32).at[0, :3].set(lin_b)

    out1 = pl.pallas_call(
        _seg_kernel,
        out_shape=jax.ShapeDtypeStruct((NP, 16), jnp.float32),
    )(x0p, srcp, dstp, w1l, b1l, w1r)

    full = lambda shape: pl.BlockSpec(shape, lambda i: (0,) * len(shape))
    out = pl.pallas_call(
        _main_kernel,
        grid=(B // R,),
        in_specs=[
            pl.BlockSpec((R, 300), lambda i: (i, 0)),
            full((300, 300)), full((300, 100)), full((1, 100)),
            full((100, 16)), full((1, 16)), full((16, NP)), full((NP, 16)),
            full((16, 16)), full((1, 16)), full((16, 16)), full((1, 16)),
            full((16, 16)), full((16, 8)), full((1, 8)),
        ],
        out_specs=pl.BlockSpec((R, 8), lambda i: (i, 0)),
        out_shape=jax.ShapeDtypeStruct((B, 8), jnp.float32),
    )(x2, w1, w2, b2, w3, b3, x0t, out1,
      w1r, b1l, w2l, b2l, w2r, lin, linb)

    return out[:, :3]


# speedup vs baseline: 15.0847x; 1.8108x over previous
import functools

import jax
import jax.numpy as jnp
from jax import lax
from jax.experimental import pallas as pl
from jax.experimental.pallas import tpu as pltpu
from jax.experimental.pallas import tpu_sc as plsc

B = 4096
N0 = 3190
E0 = 51040

NP = 3328
R = 512

NC = 2
NS = 16
ECH = 128
NCH_T = 13
EPT = NCH_T * ECH
ESC = NC * NS * EPT


def _sc_seg_kernel(x0_hbm, src_hbm, dst_hbm, zero_hbm, out_hbm,
                   src_v, dst_v, rows_v, acc_sh, sem):
    cid = lax.axis_index("c")
    sid = lax.axis_index("s")
    wid = sid * NC + cid

    @pl.when(sid == 0)
    def _():
        pltpu.sync_copy(zero_hbm, acc_sh)

    plsc.subcore_barrier()

    pltpu.sync_copy(src_hbm.at[wid], src_v)
    pltpu.sync_copy(dst_hbm.at[wid], dst_v)
    for j in range(NCH_T):
        pltpu.async_copy(x0_hbm.at[src_v.at[j]], rows_v, sem).wait()
        pltpu.sync_copy(rows_v, acc_sh.at[dst_v.at[j]], add=True)

    plsc.subcore_barrier()

    @pl.when(sid == 0)
    def _():
        pltpu.sync_copy(acc_sh, out_hbm.at[cid])


_sc_seg = functools.partial(
    pl.kernel,
    mesh=plsc.VectorSubcoreMesh(
        core_axis_name="c", subcore_axis_name="s", num_cores=NC),
    out_type=jax.ShapeDtypeStruct((NC, NP, 128), jnp.float32),
    scratch_types=[
        pltpu.VMEM((NCH_T, ECH), jnp.int32),
        pltpu.VMEM((NCH_T, ECH), jnp.int32),
        pltpu.VMEM((ECH, 128), jnp.float32),
        pltpu.VMEM_SHARED((NP, 128), jnp.float32),
        pltpu.SemaphoreType.DMA,
    ],
)(_sc_seg_kernel)


def _main_kernel(x2_ref, w1_ref, w2_ref, b2_ref, w3_ref, b3_ref, x0t_ref,
                 agg_ref, x0p_ref, w1l_ref, w1r_ref, w1rh_ref, b1l_ref, w2l_ref,
                 b2l_ref, w2r_ref, lin_ref, linb_ref, out_ref):
    acc = agg_ref[0] + agg_ref[1]
    cnt = acc[:, 10:11]
    mean = acc / jnp.maximum(cnt, 1.0)
    out1 = jnp.maximum(
        jnp.dot(mean, w1l_ref[...], preferred_element_type=jnp.float32)
        + b1l_ref[...]
        + jnp.dot(x0p_ref[...], w1r_ref[...],
                  preferred_element_type=jnp.float32),
        0.0)

    x2 = x2_ref[...]
    h1 = jnp.maximum(
        jnp.dot(x2, w1_ref[...], preferred_element_type=jnp.float32), 0.0)
    h2 = jnp.maximum(
        jnp.dot(h1, w2_ref[...], preferred_element_type=jnp.float32)
        + b2_ref[...], 0.0)
    h = jnp.maximum(
        jnp.dot(h2, w3_ref[...], preferred_element_type=jnp.float32)
        + b3_ref[...], 0.0)

    x0t = x0t_ref[...]
    a = jnp.dot(h, x0t, preferred_element_type=jnp.float32)
    norm = jnp.sqrt(jnp.sum(h * h, axis=1, keepdims=True))
    norm0 = jnp.sqrt(jnp.sum(x0t * x0t, axis=0, keepdims=True))
    a = a / norm / norm0
    col = lax.broadcasted_iota(jnp.int32, (R, NP), 1)
    neg = jnp.float32(-jnp.inf)
    a = jnp.where(col < N0, a, neg)
    a = jnp.where(jnp.isnan(a), jnp.float32(jnp.inf), a)

    idxs = []
    v0 = None
    for k in range(6):
        v = jnp.max(a, axis=1, keepdims=True)
        if k == 0:
            v0 = v
        imax = jnp.min(jnp.where(a == v, col, NP), axis=1, keepdims=True)
        idxs.append(imax)
        a = jnp.where(col == imax, neg, a)

    sel = v0 == 1.0
    s_mat = jnp.zeros((R, NP), jnp.float32)
    for k in range(5):
        pick = jnp.where(sel, idxs[k + 1], idxs[k])
        s_mat = s_mat + (col == pick).astype(jnp.float32)

    msum = jnp.dot(s_mat, out1, preferred_element_type=jnp.float32)
    mean5 = msum / 5.0

    h1h = jnp.maximum(
        b1l_ref[...]
        + jnp.dot(h, w1rh_ref[...], preferred_element_type=jnp.float32), 0.0)

    o2 = (jnp.dot(mean5, w2l_ref[...], preferred_element_type=jnp.float32)
          + b2l_ref[...]
          + jnp.dot(h1h, w2r_ref[...], preferred_element_type=jnp.float32))
    z = (jnp.dot(o2, lin_ref[...], preferred_element_type=jnp.float32)
         + linb_ref[...])
    zm = jnp.max(z, axis=1, keepdims=True)
    e = jnp.exp(z - zm)
    out_ref[...] = e / jnp.sum(e, axis=1, keepdims=True)


def _pad2(m, rows, cols):
    return jnp.zeros((rows, cols), jnp.float32).at[:m.shape[0], :m.shape[1]].set(m)


@jax.jit
def kernel(x, emb_table, fc_w, fc_b, fc2_w, fc2_b, conv1_wl, conv1_bl,
           conv1_wr, conv2_wl, conv2_bl, conv2_wr, lin_w, lin_b, x0, edge_0):
    x2 = jnp.transpose(x, (1, 0, 2)).reshape(B, 300)
    w1 = jnp.kron(jnp.eye(60, dtype=jnp.float32), emb_table)
    w2 = (fc_w.T[:, None, :, None]
          * jnp.eye(5, dtype=jnp.float32)[None, :, None, :]).reshape(300, 100)
    b2 = jnp.repeat(fc_b, 5)[None, :]
    w3 = _pad2(fc2_w.T, 100, 16)
    b3 = _pad2(fc2_b[None, :], 1, 16)

    x0p = _pad2(x0, NP, 128).at[:, 10].set(1.0)
    x0t = _pad2(x0.T, 16, NP)

    pad_e = jnp.full((2, ESC - E0), NP - 1, dtype=edge_0.dtype)
    ep = jnp.concatenate([edge_0, pad_e], axis=1)
    srcp = ep[0].reshape(NC * NS, NCH_T, ECH)
    dstp = ep[1].reshape(NC * NS, NCH_T, ECH)
    zero = jnp.zeros((NP, 128), jnp.float32)

    w1l = _pad2(conv1_wl.T, 128, 16)
    b1l = _pad2(conv1_bl[None, :], 1, 16)
    w1r = _pad2(conv1_wr.T, 128, 16)
    w1rh = _pad2(conv1_wr.T, 16, 16)
    w2l = _pad2(conv2_wl.T, 16, 16)
    b2l = _pad2(conv2_bl[None, :], 1, 16)
    w2r = _pad2(conv2_wr.T, 16, 16)
    lin = _pad2(lin_w.T, 16, 8)
    linb = jnp.full((1, 8), -1e30, jnp.float32).at[0, :3].set(lin_b)

    agg = _sc_seg(x0p, srcp, dstp, zero)

    full = lambda shape: pl.BlockSpec(shape, lambda i: (0,) * len(shape))
    out = pl.pallas_call(
        _main_kernel,
        grid=(B // R,),
        in_specs=[
            pl.BlockSpec((R, 300), lambda i: (i, 0)),
            full((300, 300)), full((300, 100)), full((1, 100)),
            full((100, 16)), full((1, 16)), full((16, NP)),
            full((NC, NP, 128)), full((NP, 128)),
            full((128, 16)), full((128, 16)), full((16, 16)), full((1, 16)),
            full((16, 16)),
            full((1, 16)), full((16, 16)), full((16, 8)), full((1, 8)),
        ],
        out_specs=pl.BlockSpec((R, 8), lambda i: (i, 0)),
        out_shape=jax.ShapeDtypeStruct((B, 8), jnp.float32),
    )(x2, w1, w2, b2, w3, b3, x0t, agg, x0p,
      w1l, w1r, w1rh, b1l, w2l, b2l, w2r, lin, linb)

    return out[:, :3]
